# Initial kernel scaffold; baseline (speedup 1.0000x reference)
import jax
import jax.numpy as jnp
from jax.experimental import pallas as pl


def kernel(x, edge_index, edge_weight, num_nodes, W0, b0, W1, b1, W2, b2):
    raise NotImplementedError("write your pallas kernel here")


# speedup vs baseline: 4.7175x; 4.7175x over previous
import functools

import jax
import jax.numpy as jnp
from jax import lax
from jax.experimental import pallas as pl
from jax.experimental.pallas import tpu as pltpu
from jax.experimental.pallas import tpu_sc as plsc

N = 10000
NPAD = 10240
E = 320000
D = 128
LANES = 16
EB = 128
NB = E // EB
NC = 2
NS = 16
NW = NC * NS
ROWS_PER_TILE = NPAD // NS
ZCHUNK = 128


def _hop_body(x_hbm, src_hbm, dst_hbm, w_hbm, out_hbm,
              col_v, dst_v, w_v, rows_v, zero_v, acc_sh, sem):
    cid = lax.axis_index("c")
    sid = lax.axis_index("s")
    wid = sid * NC + cid

    def zfill(r, _):
        for f in range(D // LANES):
            zero_v[r, pl.ds(f * LANES, LANES)] = jnp.zeros((LANES,), jnp.float32)
        return _
    lax.fori_loop(0, ZCHUNK, zfill, None)
    base_row = sid * ROWS_PER_TILE
    for z in range(ROWS_PER_TILE // ZCHUNK):
        pltpu.sync_copy(zero_v, acc_sh.at[pl.ds(base_row + z * ZCHUNK, ZCHUNK)])
    plsc.subcore_barrier()

    nb = (NB - wid + NW - 1) // NW

    def batch(b, _):
        base = (wid + b * NW) * EB
        pltpu.sync_copy(src_hbm.at[pl.ds(base, EB)], col_v)
        pltpu.sync_copy(dst_hbm.at[pl.ds(base, EB)], dst_v)
        pltpu.sync_copy(w_hbm.at[pl.ds(base, EB)], w_v)
        pltpu.async_copy(x_hbm.at[col_v], rows_v, sem).wait()

        def scale(g, _):
            wchunk = w_v[pl.ds(g * LANES, LANES)]
            for k in range(LANES):
                idx = jnp.full((LANES,), k, jnp.int32)
                wsplat = wchunk.at[idx].get(mode="promise_in_bounds")
                e = g * LANES + k
                for f in range(D // LANES):
                    sl = pl.ds(f * LANES, LANES)
                    rows_v[e, sl] = rows_v[e, sl] * wsplat
            return _
        lax.fori_loop(0, EB // LANES, scale, None)

        pltpu.sync_copy(rows_v, acc_sh.at[dst_v], add=True)
        return _
    lax.fori_loop(0, nb, batch, None)

    plsc.subcore_barrier()
    pltpu.sync_copy(acc_sh.at[pl.ds(base_row, ROWS_PER_TILE)],
                    out_hbm.at[cid, pl.ds(base_row, ROWS_PER_TILE)])


@jax.jit
def _hop(x, src, dst, w):
    mesh = plsc.VectorSubcoreMesh(core_axis_name="c", subcore_axis_name="s")
    f = pl.kernel(
        _hop_body,
        out_type=jax.ShapeDtypeStruct((NC, NPAD, D), jnp.float32),
        mesh=mesh,
        scratch_types=[
            pltpu.VMEM((EB,), jnp.int32),
            pltpu.VMEM((EB,), jnp.int32),
            pltpu.VMEM((EB,), jnp.float32),
            pltpu.VMEM((EB, D), jnp.float32),
            pltpu.VMEM((ZCHUNK, D), jnp.float32),
            pltpu.VMEM_SHARED((NPAD, D), jnp.float32),
            pltpu.SemaphoreType.DMA,
        ],
    )
    return f(x, src, dst, w)


def _tc_add_body(a_ref, o_ref):
    o_ref[...] = a_ref[0] + a_ref[1]


@jax.jit
def _tc_add(a):
    blk = 1000
    return pl.pallas_call(
        _tc_add_body,
        grid=(N // blk,),
        in_specs=[pl.BlockSpec((NC, blk, D), lambda i: (0, i, 0))],
        out_specs=pl.BlockSpec((blk, D), lambda i: (i, 0)),
        out_shape=jax.ShapeDtypeStruct((N, D), jnp.float32),
    )(a)


def _tc_final_body(x_ref, h1_ref, b_ref, w0_ref, b0_ref, w1_ref, b1_ref,
                   w2_ref, b2_ref, o_ref):
    h2 = b_ref[0] + b_ref[1]
    dn = (((1,), (1,)), ((), ()))
    o_ref[:, 0:D] = lax.dot_general(
        x_ref[...], w0_ref[...], dn, preferred_element_type=jnp.float32) + b0_ref[...]
    o_ref[:, D:2 * D] = lax.dot_general(
        h1_ref[...], w1_ref[...], dn, preferred_element_type=jnp.float32) + b1_ref[...]
    o_ref[:, 2 * D:3 * D] = lax.dot_general(
        h2, w2_ref[...], dn, preferred_element_type=jnp.float32) + b2_ref[...]


@jax.jit
def _tc_final(x, h1, b_parts, W0, b0, W1, b1, W2, b2):
    blk = 1000
    wspec = pl.BlockSpec((D, D), lambda i: (0, 0))
    bspec = pl.BlockSpec((1, D), lambda i: (0, 0))
    return pl.pallas_call(
        _tc_final_body,
        grid=(N // blk,),
        in_specs=[
            pl.BlockSpec((blk, D), lambda i: (i, 0)),
            pl.BlockSpec((blk, D), lambda i: (i, 0)),
            pl.BlockSpec((NC, blk, D), lambda i: (0, i, 0)),
            wspec, bspec, wspec, bspec, wspec, bspec,
        ],
        out_specs=pl.BlockSpec((blk, 3 * D), lambda i: (i, 0)),
        out_shape=jax.ShapeDtypeStruct((N, 3 * D), jnp.float32),
    )(x, h1, b_parts, W0, b0.reshape(1, D), W1, b1.reshape(1, D),
      W2, b2.reshape(1, D))


def kernel(x, edge_index, edge_weight, num_nodes, W0, b0, W1, b1, W2, b2):
    src = edge_index[1]
    dst = edge_index[0]
    a_parts = _hop(x, src, dst, edge_weight)
    h1 = _tc_add(a_parts)
    b_parts = _hop(h1, src, dst, edge_weight)
    return _tc_final(x, h1, b_parts, W0, b0, W1, b1, W2, b2)
